# baseline (speedup 1.0000x reference)
import functools

import jax
import jax.numpy as jnp
from jax import lax
from jax.experimental import pallas as pl
from jax.experimental.pallas import tpu as pltpu
from jax.experimental.pallas import tpu_sc as plsc

_EMB_DIM = 64
_LANES = 16
_CHUNK = 256
_B_POS = 16384
_B_NEG = 81920


def _row_dots(u_rows, v_rows, scores, s_base):
    lanes = lax.iota(jnp.int32, _LANES)
    masks = [(lanes & d) == 0 for d in (8, 4, 2, 1)]
    perms = [lanes ^ d for d in (8, 4, 2, 1)]

    def combine(a, b, lvl):
        m = masks[lvl]
        pa = jnp.take(a, perms[lvl])
        pb = jnp.take(b, perms[lvl])
        return jnp.where(m, a, pb) + jnp.where(m, pa, b)

    def group_body(g, _):
        row0 = g * _LANES
        stack = []
        for r in range(_LANES):
            acc = jnp.zeros((_LANES,), jnp.float32)
            row = row0 + r
            for a in range(4):
                u = u_rows[row, pl.ds(a * 16, 16)]
                v = v_rows[row, pl.ds(a * 16, 16)]
                acc = acc + u * v
            node, lvl = acc, 0
            while stack and stack[-1][0] == lvl:
                _, prev = stack.pop()
                node = combine(prev, node, lvl)
                lvl += 1
            stack.append((lvl, node))
        scores[pl.ds(s_base + row0, _LANES)] = stack[0][1]
        return 0

    lax.fori_loop(0, _CHUNK // _LANES, group_body, 0)


def _make_sc_dots():
    info = plsc.get_sparse_core_info()
    nc, ns = info.num_cores, info.num_subcores
    nw = nc * ns
    pos_per_w = _B_POS // nw
    neg_per_w = _B_NEG // nw
    tot_per_w = pos_per_w + neg_per_w

    mesh = plsc.VectorSubcoreMesh(core_axis_name="c", subcore_axis_name="s")

    @functools.partial(
        pl.kernel,
        out_type=jax.ShapeDtypeStruct((nw, _LANES), jnp.float32),
        mesh=mesh,
        compiler_params=pltpu.CompilerParams(
            use_tc_tiling_on_sc=False, needs_layout_passes=False),
        scratch_types=[
            pltpu.VMEM((tot_per_w,), jnp.int32),
            pltpu.VMEM((tot_per_w,), jnp.int32),
            pltpu.VMEM((_CHUNK, _EMB_DIM), jnp.float32),
            pltpu.VMEM((_CHUNK, _EMB_DIM), jnp.float32),
            pltpu.VMEM((_CHUNK, _EMB_DIM), jnp.float32),
            pltpu.VMEM((_CHUNK, _EMB_DIM), jnp.float32),
            pltpu.VMEM((_CHUNK, _EMB_DIM), jnp.float32),
            pltpu.VMEM((_CHUNK, _EMB_DIM), jnp.float32),
            pltpu.VMEM((tot_per_w,), jnp.float32),
            pltpu.VMEM((_LANES,), jnp.int32),
            pltpu.SemaphoreType.DMA,
            pltpu.SemaphoreType.DMA,
            pltpu.SemaphoreType.DMA,
        ],
    )
    def sc_dots(pos_u_hbm, pos_v_hbm, neg_u_hbm, neg_v_hbm, ord_hbm,
                u_hbm, v_hbm, part_out,
                iu_all, iv_all, u0, v0, u1, v1, u2, v2, scores, ord_v,
                sem0, sem1, sem2):
        wid = lax.axis_index("s") * nc + lax.axis_index("c")

        pltpu.sync_copy(ord_hbm, ord_v)
        is1 = jnp.max(ord_v[...]) == 1

        pltpu.sync_copy(pos_u_hbm.at[pl.ds(wid * pos_per_w, pos_per_w)],
                        iu_all.at[pl.ds(0, pos_per_w)])
        pltpu.sync_copy(pos_v_hbm.at[pl.ds(wid * pos_per_w, pos_per_w)],
                        iv_all.at[pl.ds(0, pos_per_w)])
        pltpu.sync_copy(neg_u_hbm.at[pl.ds(wid * neg_per_w, neg_per_w)],
                        iu_all.at[pl.ds(pos_per_w, neg_per_w)])
        pltpu.sync_copy(neg_v_hbm.at[pl.ds(wid * neg_per_w, neg_per_w)],
                        iv_all.at[pl.ds(pos_per_w, neg_per_w)])

        n_pos = pos_per_w // _CHUNK
        n_tot = tot_per_w // _CHUNK

        def issue(c, u_buf, v_buf, sem):
            for h in range(2):
                off = c * _CHUNK + h * 128
                iu = iu_all.at[pl.ds(off, 128)]
                iv = iv_all.at[pl.ds(off, 128)]
                dst_u = u_buf.at[pl.ds(h * 128, 128)]
                dst_v = v_buf.at[pl.ds(h * 128, 128)]
                pltpu.async_copy(u_hbm.at[iu], dst_u, sem)
                from_u = jnp.logical_and(c < n_pos, is1)

                @pl.when(from_u)
                def _():
                    pltpu.async_copy(u_hbm.at[iv], dst_v, sem)

                @pl.when(jnp.logical_not(from_u))
                def _():
                    pltpu.async_copy(v_hbm.at[iv], dst_v, sem)

        def run(c, u_buf, v_buf, sem):
            iu = iu_all.at[pl.ds(0, 128)]
            for h in range(2):
                dst_u = u_buf.at[pl.ds(h * 128, 128)]
                dst_v = v_buf.at[pl.ds(h * 128, 128)]
                pltpu.make_async_copy(u_hbm.at[iu], dst_u, sem).wait()
                pltpu.make_async_copy(u_hbm.at[iu], dst_v, sem).wait()
            _row_dots(u_buf, v_buf, scores, c * _CHUNK)

        issue(0, u0, v0, sem0)
        issue(1, u1, v1, sem1)

        def ring_body(p, _):
            c0 = 3 * p
            issue(c0 + 2, u2, v2, sem2)
            run(c0, u0, v0, sem0)

            @pl.when(c0 + 3 < n_tot)
            def _():
                issue(c0 + 3, u0, v0, sem0)

            run(c0 + 1, u1, v1, sem1)

            @pl.when(c0 + 4 < n_tot)
            def _():
                issue(c0 + 4, u1, v1, sem1)

            run(c0 + 2, u2, v2, sem2)
            return 0

        lax.fori_loop(0, n_tot // 3, ring_body, 0)

        def logsig_sum(base, count, sign):
            def vec_body(i, acc):
                x = scores[pl.ds(base + i * _LANES, _LANES)] * sign
                w = 1.0 + jnp.exp(-jnp.abs(x))
                t = (w - 1.0) / (w + 1.0)
                t2 = t * t
                ln_w = 2.0 * t * (1.0 + t2 * (1.0 / 3.0 + t2 * (
                    1.0 / 5.0 + t2 * (1.0 / 7.0 + t2 * (1.0 / 9.0)))))
                return acc + (ln_w - jnp.minimum(x, 0.0))

            return lax.fori_loop(0, count // _LANES, vec_body,
                                 jnp.zeros((_LANES,), jnp.float32), unroll=4)

        part = logsig_sum(0, pos_per_w, 1.0) + logsig_sum(
            pos_per_w, neg_per_w, -1.0)
        scores[pl.ds(0, _LANES)] = part
        pltpu.sync_copy(scores.at[pl.ds(0, _LANES)], part_out.at[wid])

    return sc_dots


def kernel(pos_u, pos_v, neg_u, neg_v, order, U, V):
    ord_vec = jnp.full((_LANES,), order, dtype=jnp.int32)
    sc_dots = _make_sc_dots()
    partials = sc_dots(
        pos_u.astype(jnp.int32), pos_v.astype(jnp.int32),
        neg_u.astype(jnp.int32), neg_v.astype(jnp.int32),
        ord_vec, U, V)
    return jnp.sum(partials)
